# baseline (speedup 1.0000x reference)
import functools

import jax
import jax.numpy as jnp
from jax import lax
from jax.experimental import pallas as pl
from jax.experimental.pallas import tpu as pltpu
from jax.experimental.pallas import tpu_sc as plsc

N_CLS = 1000
D = 128
B = 16384
PROTO_M = 0.999
TEMP = 0.1
BASE_TEMP = 0.1

NCLS_PAD = 1024
NW = 32
K = NCLS_PAD // NW
NV = D // 16
CH = 128
HK = K // 2
NT = 16
SLICE = B // NT
NVREG = SLICE // 16
QLEN = B + 2 * CH
BASE_LEN = NCLS_PAD + 64


def _ema_kernel(feat_hbm, lbl_hbm, protos_hbm, out_hbm,
                protos_v, lblv, cntv, matv, basev, curv, posb, valb,
                idxa0, idxa1, idxb0, idxb1, rowsa0, rowsa1, rowsb0, rowsb1,
                cntmat_sh, queue_sh, sema0, sema1, semb0, semb1):
    sid = lax.axis_index("s")
    wid = sid * 2 + lax.axis_index("c")
    c0 = wid * K
    lane = lax.iota(jnp.int32, 16)
    zeros16 = jnp.zeros((16,), jnp.int32)
    ones16 = jnp.ones((16,), jnp.int32)

    pltpu.sync_copy(lbl_hbm.at[pl.ds(sid * SLICE, SLICE)], lblv)

    def zero_body(i, _):
        cntv[pl.ds(i * 16, 16)] = zeros16
        return _
    lax.fori_loop(0, NCLS_PAD // 16, zero_body, 0)

    def hist_body(i, _):
        lv = lblv[pl.ds(i * 16, 16)]
        plsc.addupdate_scatter(cntv, [lv], ones16)
        return _
    lax.fori_loop(0, NVREG, hist_body, 0)

    pltpu.sync_copy(cntv, cntmat_sh.at[sid])
    plsc.subcore_barrier()
    pltpu.sync_copy(cntmat_sh, matv)

    def colsum_body(j, _):
        tot = zeros16
        pre = zeros16
        for t in range(NT):
            row = matv[t, pl.ds(j * 16, 16)]
            tot = tot + row
            pre = pre + row * (jnp.int32(t) < sid).astype(jnp.int32)
        cntv[pl.ds(j * 16, 16)] = tot
        curv[pl.ds(j * 16, 16)] = pre
        return _
    lax.fori_loop(0, NCLS_PAD // 16, colsum_body, 0)

    def cumsum_body(j, run):
        chv = cntv[pl.ds(j * 16, 16)]
        inc = plsc.cumsum(chv)
        ex = inc - chv + jnp.full((16,), run, jnp.int32)
        basev[pl.ds(j * 16, 16)] = ex
        curv[pl.ds(j * 16, 16)] = curv[pl.ds(j * 16, 16)] + ex
        return run + inc[15]
    total = lax.fori_loop(0, NCLS_PAD // 16, cumsum_body, jnp.int32(0))
    for j in range(NCLS_PAD // 16, BASE_LEN // 16):
        basev[pl.ds(j * 16, 16)] = jnp.full((16,), total, jnp.int32)

    def place_body(i, _):
        lv = lblv[pl.ds(i * 16, 16)]
        cnt, _last = plsc.scan_count(lv)
        g = plsc.load_gather(curv, [lv])
        pos = g + cnt - 1
        r = lax.div(i, 8)
        col = 16 * lax.rem(i, 8)
        posb[r, pl.ds(col, 16)] = pos
        valb[r, pl.ds(col, 16)] = sid * SLICE + i * 16 + lane
        plsc.addupdate_scatter(curv, [lv], ones16)
        return _
    lax.fori_loop(0, NVREG, place_body, 0)

    for jj in range(8):
        pltpu.async_copy(valb.at[jj], queue_sh.at[posb.at[jj]], sema0)
    for jj in range(8):
        pltpu.make_async_copy(valb.at[jj], queue_sh.at[posb.at[jj]],
                              sema0).wait()

    @pl.when(sid == 0)
    def _():
        def zq_body(i, _):
            idxa0[pl.ds(i * 16, 16)] = zeros16
            return _
        lax.fori_loop(0, CH // 16, zq_body, 0)
        pltpu.sync_copy(idxa0, queue_sh.at[pl.ds(B, CH)])
        pltpu.sync_copy(idxa0, queue_sh.at[pl.ds(B + CH, CH)])
    plsc.subcore_barrier()

    pltpu.sync_copy(protos_hbm.at[pl.ds(c0, K)], protos_v)
    bfly = [lax.bitwise_xor(lane, jnp.int32(s)) for s in (8, 4, 2, 1)]
    idxs = ((idxa0, idxa1), (idxb0, idxb1))
    rows = ((rowsa0, rowsa1), (rowsb0, rowsb1))
    sems = ((sema0, sema1), (semb0, semb1))

    def off_at(i):
        return basev[pl.ds(c0 + i, 16)][0]

    meta = []
    for st in range(2):
        s0 = off_at(st * HK)
        s1 = off_at(st * HK + HK)
        al = pl.multiple_of(lax.bitwise_and(s0, -16), 16)
        nwin = lax.div(s1 - al + (CH - 1), CH)
        meta.append((s0, s1, al, nwin))
    nwin_max = jnp.maximum(meta[0][3], meta[1][3])

    def prefetch(st, t, buf):
        al = meta[st][2]
        start = pl.multiple_of(al + t * CH, 16)
        pltpu.sync_copy(queue_sh.at[pl.ds(start, CH)], idxs[st][buf])
        pltpu.async_copy(feat_hbm.at[idxs[st][buf]], rows[st][buf],
                         sems[st][buf])

    for st in range(2):
        @pl.when(meta[st][3] > 0)
        def _(st=st):
            prefetch(st, 0, 0)

    def ema_step(st, buf, r, t, st_carry):
        s0, s1, al, _ = meta[st]
        j = al + t * CH + r
        active = jnp.logical_and(j >= s0, j < s1)

        def sw_true(ci, bnd, *p):
            for v in range(NV):
                protos_v[ci, pl.ds(v * 16, 16)] = p[v]
            ci = lax.while_loop(lambda a: off_at(a + 1) <= j,
                                lambda a: a + 1, ci)
            return (ci, off_at(ci + 1)) + tuple(
                protos_v[ci, pl.ds(v * 16, 16)] for v in range(NV))

        def sw_false(ci, bnd, *p):
            return (ci, bnd) + tuple(p)

        st_carry = lax.cond(
            jnp.logical_and(active, j >= st_carry[1]),
            sw_true, sw_false, *st_carry)
        p = st_carry[2:]
        q = [PROTO_M * p[v]
             + (1.0 - PROTO_M) * rows[st][buf][r, pl.ds(v * 16, 16)]
             for v in range(NV)]
        d = [q[2 * v] * q[2 * v] + q[2 * v + 1] * q[2 * v + 1]
             for v in range(NV // 2)]
        d = [d[0] + d[1], d[2] + d[3]]
        acc = d[0] + d[1]
        for bf in bfly:
            acc = acc + acc.at[bf].get(mode="promise_in_bounds")
        n2v = jnp.maximum(acc, jnp.float32(1e-24))
        y = plsc.bitcast(
            jnp.int32(0x5F3759DF)
            - lax.shift_right_arithmetic(
                plsc.bitcast(n2v, jnp.int32), jnp.int32(1)),
            jnp.float32)
        h = jnp.float32(0.5) * n2v
        for _ in range(2):
            y = y * (jnp.float32(1.5) - h * y * y)
        amask = jnp.full((16,), active, jnp.bool_)
        return st_carry[:2] + tuple(
            jnp.where(amask, q[v] * y, p[v]) for v in range(NV))

    def do_windows(t, buf, carry):
        for st in range(2):
            @pl.when(t + 1 < meta[st][3])
            def _(st=st):
                prefetch(st, t + 1, 1 - buf)

            @pl.when(t < meta[st][3])
            def _(st=st):
                pltpu.make_async_copy(feat_hbm.at[idxs[st][buf]],
                                      rows[st][buf], sems[st][buf]).wait()

        def samp_body(r, carry):
            ca = ema_step(0, buf, r, t, carry[:2 + NV])
            cb = ema_step(1, buf, r, t, carry[2 + NV:])
            return ca + cb

        return lax.fori_loop(0, CH, samp_body, carry)

    def pair_body(g, carry):
        carry = do_windows(2 * g, 0, carry)
        carry = do_windows(2 * g + 1, 1, carry)
        return carry

    def init_stream(st):
        s0 = meta[st][0]
        ci = lax.while_loop(
            lambda a: jnp.logical_and(a < st * HK + HK,
                                      off_at(a + 1) <= s0),
            lambda a: a + 1, st * HK)
        ci = jnp.minimum(ci, st * HK + HK - 1)
        return (ci, off_at(ci + 1)) + tuple(
            protos_v[ci, pl.ds(v * 16, 16)] for v in range(NV))

    carry = init_stream(0) + init_stream(1)
    carry = lax.fori_loop(0, lax.div(nwin_max + 1, 2), pair_body, carry)

    for st in range(2):
        @pl.when(meta[st][1] > meta[st][0])
        def _(st=st):
            st_carry = carry[st * (2 + NV):(st + 1) * (2 + NV)]
            for v in range(NV):
                protos_v[st_carry[0], pl.ds(v * 16, 16)] = st_carry[2 + v]

    pltpu.sync_copy(protos_v, out_hbm.at[pl.ds(c0, K)])


def _loss_kernel(p_ref, o_ref):
    p = p_ref[...]
    s = lax.dot_general(p, p, (((1,), (1,)), ((), ())),
                        preferred_element_type=jnp.float32) * (1.0 / TEMP)
    row = lax.broadcasted_iota(jnp.int32, (NCLS_PAD, NCLS_PAD), 0)
    col = lax.broadcasted_iota(jnp.int32, (NCLS_PAD, NCLS_PAD), 1)
    neg = jnp.logical_and(col < N_CLS, col != row)
    e = jnp.where(neg, jnp.exp(s), 0.0)
    rs = jnp.sum(e, axis=1)
    mpn = jnp.log(rs / jnp.float32(N_CLS - 1))
    rvalid = lax.broadcasted_iota(jnp.int32, (NCLS_PAD, 1), 0) < N_CLS
    total = jnp.sum(jnp.where(rvalid[:, 0], mpn, 0.0))
    loss = (TEMP / BASE_TEMP) * total / jnp.float32(N_CLS)
    o_ref[...] = jnp.full((8, 128), loss, jnp.float32)


@jax.jit
def kernel(features, labels, prototypes):
    labels = labels.astype(jnp.int32)
    protos_pad = jnp.pad(prototypes, ((0, NCLS_PAD - N_CLS), (0, 0)))

    ema = pl.kernel(
        _ema_kernel,
        out_type=jax.ShapeDtypeStruct((NCLS_PAD, D), jnp.float32),
        mesh=plsc.VectorSubcoreMesh(core_axis_name="c", subcore_axis_name="s"),
        compiler_params=pltpu.CompilerParams(needs_layout_passes=False),
        scratch_types=[
            pltpu.VMEM((K, D), jnp.float32),
            pltpu.VMEM((SLICE,), jnp.int32),
            pltpu.VMEM((NCLS_PAD,), jnp.int32),
            pltpu.VMEM((NT, NCLS_PAD), jnp.int32),
            pltpu.VMEM((BASE_LEN,), jnp.int32),
            pltpu.VMEM((NCLS_PAD,), jnp.int32),
            pltpu.VMEM((8, 128), jnp.int32),
            pltpu.VMEM((8, 128), jnp.int32),
            pltpu.VMEM((CH,), jnp.int32),
            pltpu.VMEM((CH,), jnp.int32),
            pltpu.VMEM((CH,), jnp.int32),
            pltpu.VMEM((CH,), jnp.int32),
            pltpu.VMEM((CH, D), jnp.float32),
            pltpu.VMEM((CH, D), jnp.float32),
            pltpu.VMEM((CH, D), jnp.float32),
            pltpu.VMEM((CH, D), jnp.float32),
            pltpu.VMEM_SHARED((NT, NCLS_PAD), jnp.int32),
            pltpu.VMEM_SHARED((QLEN,), jnp.int32),
            pltpu.SemaphoreType.DMA,
            pltpu.SemaphoreType.DMA,
            pltpu.SemaphoreType.DMA,
            pltpu.SemaphoreType.DMA,
        ],
    )
    protos_new = ema(features, labels, protos_pad)

    loss2d = pl.pallas_call(
        _loss_kernel,
        out_shape=jax.ShapeDtypeStruct((8, 128), jnp.float32),
    )(protos_new)
    return loss2d[0, 0]


# speedup vs baseline: 1.1224x; 1.1224x over previous
import functools

import jax
import jax.numpy as jnp
from jax import lax
from jax.experimental import pallas as pl
from jax.experimental.pallas import tpu as pltpu
from jax.experimental.pallas import tpu_sc as plsc

N_CLS = 1000
D = 128
B = 16384
PROTO_M = 0.999
TEMP = 0.1
BASE_TEMP = 0.1

NCLS_PAD = 1024
NW = 32
K = NCLS_PAD // NW
NV = D // 16
CH = 256
NT = 16
SLICE = B // NT
NVREG = SLICE // 16
QLEN = B + 2 * CH
BASE_LEN = NCLS_PAD + 64


def _ema_kernel(feat_hbm, lbl_hbm, protos_hbm, out_hbm,
                protos_v, lblv, cntv, matv, basev, curv, posb, valb,
                idx0, idx1, rows0, rows1, cntmat_sh, queue_sh,
                sem0, sem1):
    sid = lax.axis_index("s")
    wid = sid * 2 + lax.axis_index("c")
    c0 = wid * K
    lane = lax.iota(jnp.int32, 16)
    zeros16 = jnp.zeros((16,), jnp.int32)
    ones16 = jnp.ones((16,), jnp.int32)

    pltpu.sync_copy(lbl_hbm.at[pl.ds(sid * SLICE, SLICE)], lblv)

    def zero_body(i, _):
        cntv[pl.ds(i * 16, 16)] = zeros16
        return _
    lax.fori_loop(0, NCLS_PAD // 16, zero_body, 0)

    def hist_body(i, _):
        lv = lblv[pl.ds(i * 16, 16)]
        plsc.addupdate_scatter(cntv, [lv], ones16)
        return _
    lax.fori_loop(0, NVREG, hist_body, 0)

    pltpu.sync_copy(cntv, cntmat_sh.at[sid])
    plsc.subcore_barrier()
    pltpu.sync_copy(cntmat_sh, matv)

    def colsum_body(j, _):
        tot = zeros16
        pre = zeros16
        for t in range(NT):
            row = matv[t, pl.ds(j * 16, 16)]
            tot = tot + row
            pre = pre + row * (jnp.int32(t) < sid).astype(jnp.int32)
        cntv[pl.ds(j * 16, 16)] = tot
        curv[pl.ds(j * 16, 16)] = pre
        return _
    lax.fori_loop(0, NCLS_PAD // 16, colsum_body, 0)

    def cumsum_body(j, run):
        chv = cntv[pl.ds(j * 16, 16)]
        inc = plsc.cumsum(chv)
        ex = inc - chv + jnp.full((16,), run, jnp.int32)
        basev[pl.ds(j * 16, 16)] = ex
        curv[pl.ds(j * 16, 16)] = curv[pl.ds(j * 16, 16)] + ex
        return run + inc[15]
    total = lax.fori_loop(0, NCLS_PAD // 16, cumsum_body, jnp.int32(0))
    for j in range(NCLS_PAD // 16, BASE_LEN // 16):
        basev[pl.ds(j * 16, 16)] = jnp.full((16,), total, jnp.int32)

    def place_body(i, _):
        lv = lblv[pl.ds(i * 16, 16)]
        cnt, _last = plsc.scan_count(lv)
        g = plsc.load_gather(curv, [lv])
        pos = g + cnt - 1
        r = lax.div(i, 8)
        col = 16 * lax.rem(i, 8)
        posb[r, pl.ds(col, 16)] = pos
        valb[r, pl.ds(col, 16)] = sid * SLICE + i * 16 + lane
        plsc.addupdate_scatter(curv, [lv], ones16)
        return _
    lax.fori_loop(0, NVREG, place_body, 0)

    for jj in range(8):
        pltpu.async_copy(valb.at[jj], queue_sh.at[posb.at[jj]], sem0)
    for jj in range(8):
        pltpu.make_async_copy(valb.at[jj], queue_sh.at[posb.at[jj]],
                              sem0).wait()

    @pl.when(sid == 0)
    def _():
        def zq_body(i, _):
            idx0[pl.ds(i * 16, 16)] = zeros16
            return _
        lax.fori_loop(0, CH // 16, zq_body, 0)
        pltpu.sync_copy(idx0, queue_sh.at[pl.ds(B, CH)])
        pltpu.sync_copy(idx0, queue_sh.at[pl.ds(B + CH, CH)])
    plsc.subcore_barrier()

    pltpu.sync_copy(protos_hbm.at[pl.ds(c0, K)], protos_v)
    bfly = [lax.bitwise_xor(lane, jnp.int32(s)) for s in (8, 4, 2, 1)]
    sems = (sem0, sem1)
    idxs = (idx0, idx1)
    rows = (rows0, rows1)

    def off_at(i):
        return basev[pl.ds(c0 + i, 16)][0]

    s0 = off_at(0)
    s1 = off_at(K)
    al = pl.multiple_of(lax.bitwise_and(s0, -16), 16)
    nwin = lax.div(s1 - al + (CH - 1), CH)

    def win_start(t):
        return pl.multiple_of(al + t * CH, 16)

    def prefetch(t, buf):
        pltpu.sync_copy(queue_sh.at[pl.ds(win_start(t), CH)], idxs[buf])
        pltpu.async_copy(feat_hbm.at[idxs[buf]], rows[buf], sems[buf])

    @pl.when(nwin > 0)
    def _():
        prefetch(0, 0)

    def do_window(t, buf, cur_p):
        @pl.when(t + 1 < nwin)
        def _():
            prefetch(t + 1, 1 - buf)

        @pl.when(t < nwin)
        def _():
            pltpu.make_async_copy(feat_hbm.at[idxs[buf]],
                                  rows[buf], sems[buf]).wait()

        base = al + t * CH
        lo = jnp.maximum(s0 - base, 0)
        hi = jnp.minimum(s1 - base, CH)

        def samp_body(r, cur_p):
            j = base + r

            def sw_true(ci, bnd, *p):
                for v in range(NV):
                    protos_v[ci, pl.ds(v * 16, 16)] = p[v]
                ci = lax.while_loop(lambda a: off_at(a + 1) <= j,
                                    lambda a: a + 1, ci)
                return (ci, off_at(ci + 1)) + tuple(
                    protos_v[ci, pl.ds(v * 16, 16)] for v in range(NV))

            def sw_false(ci, bnd, *p):
                return (ci, bnd) + tuple(p)

            cur_p = lax.cond(j >= cur_p[1], sw_true, sw_false, *cur_p)
            p = cur_p[2:]
            q = [PROTO_M * p[v]
                 + (1.0 - PROTO_M) * rows[buf][r, pl.ds(v * 16, 16)]
                 for v in range(NV)]
            d = [q[2 * v] * q[2 * v] + q[2 * v + 1] * q[2 * v + 1]
                 for v in range(NV // 2)]
            d = [d[0] + d[1], d[2] + d[3]]
            acc = d[0] + d[1]
            n2 = plsc.cumsum(acc)[15]
            n2v = jnp.maximum(jnp.full((16,), n2, jnp.float32),
                              jnp.float32(1e-24))
            y = plsc.bitcast(
                jnp.int32(0x5F3759DF)
                - lax.shift_right_arithmetic(
                    plsc.bitcast(n2v, jnp.int32), jnp.int32(1)),
                jnp.float32)
            h = jnp.float32(0.5) * n2v
            for _ in range(2):
                y = y * (jnp.float32(1.5) - h * y * y)
            return cur_p[:2] + tuple(q[v] * y for v in range(NV))

        return lax.fori_loop(lo, hi, samp_body, cur_p)

    def pair_body(g, cur_p):
        cur_p = do_window(2 * g, 0, cur_p)
        cur_p = do_window(2 * g + 1, 1, cur_p)
        return cur_p

    ci0 = lax.while_loop(
        lambda a: jnp.logical_and(a < K, off_at(a + 1) <= s0),
        lambda a: a + 1, 0)
    ci0 = jnp.minimum(ci0, K - 1)
    cur_p = (ci0, off_at(ci0 + 1)) + tuple(
        protos_v[ci0, pl.ds(v * 16, 16)] for v in range(NV))
    cur_p = lax.fori_loop(0, lax.div(nwin + 1, 2), pair_body, cur_p)

    @pl.when(s1 > s0)
    def _():
        for v in range(NV):
            protos_v[cur_p[0], pl.ds(v * 16, 16)] = cur_p[2 + v]

    pltpu.sync_copy(protos_v, out_hbm.at[pl.ds(c0, K)])


def _loss_kernel(p_ref, o_ref):
    p = p_ref[...]
    s = lax.dot_general(p, p, (((1,), (1,)), ((), ())),
                        preferred_element_type=jnp.float32) * (1.0 / TEMP)
    row = lax.broadcasted_iota(jnp.int32, (NCLS_PAD, NCLS_PAD), 0)
    col = lax.broadcasted_iota(jnp.int32, (NCLS_PAD, NCLS_PAD), 1)
    neg = jnp.logical_and(col < N_CLS, col != row)
    e = jnp.where(neg, jnp.exp(s), 0.0)
    rs = jnp.sum(e, axis=1)
    mpn = jnp.log(rs / jnp.float32(N_CLS - 1))
    rvalid = lax.broadcasted_iota(jnp.int32, (NCLS_PAD, 1), 0) < N_CLS
    total = jnp.sum(jnp.where(rvalid[:, 0], mpn, 0.0))
    loss = (TEMP / BASE_TEMP) * total / jnp.float32(N_CLS)
    o_ref[...] = jnp.full((8, 128), loss, jnp.float32)


@jax.jit
def kernel(features, labels, prototypes):
    labels = labels.astype(jnp.int32)
    protos_pad = jnp.pad(prototypes, ((0, NCLS_PAD - N_CLS), (0, 0)))

    ema = pl.kernel(
        _ema_kernel,
        out_type=jax.ShapeDtypeStruct((NCLS_PAD, D), jnp.float32),
        mesh=plsc.VectorSubcoreMesh(core_axis_name="c", subcore_axis_name="s"),
        compiler_params=pltpu.CompilerParams(needs_layout_passes=False),
        scratch_types=[
            pltpu.VMEM((K, D), jnp.float32),
            pltpu.VMEM((SLICE,), jnp.int32),
            pltpu.VMEM((NCLS_PAD,), jnp.int32),
            pltpu.VMEM((NT, NCLS_PAD), jnp.int32),
            pltpu.VMEM((BASE_LEN,), jnp.int32),
            pltpu.VMEM((NCLS_PAD,), jnp.int32),
            pltpu.VMEM((8, 128), jnp.int32),
            pltpu.VMEM((8, 128), jnp.int32),
            pltpu.VMEM((CH,), jnp.int32),
            pltpu.VMEM((CH,), jnp.int32),
            pltpu.VMEM((CH, D), jnp.float32),
            pltpu.VMEM((CH, D), jnp.float32),
            pltpu.VMEM_SHARED((NT, NCLS_PAD), jnp.int32),
            pltpu.VMEM_SHARED((QLEN,), jnp.int32),
            pltpu.SemaphoreType.DMA,
            pltpu.SemaphoreType.DMA,
        ],
    )
    protos_new = ema(features, labels, protos_pad)

    loss2d = pl.pallas_call(
        _loss_kernel,
        out_shape=jax.ShapeDtypeStruct((8, 128), jnp.float32),
    )(protos_new)
    return loss2d[0, 0]


# speedup vs baseline: 1.1563x; 1.0302x over previous
import functools

import jax
import jax.numpy as jnp
from jax import lax
from jax.experimental import pallas as pl
from jax.experimental.pallas import tpu as pltpu
from jax.experimental.pallas import tpu_sc as plsc

N_CLS = 1000
D = 128
B = 16384
PROTO_M = 0.999
TEMP = 0.1
BASE_TEMP = 0.1

NCLS_PAD = 1024
NW = 32
K = NCLS_PAD // NW
NV = D // 16
CH = 256
NT = 16
SLICE = B // NT
NVREG = SLICE // 16
QLEN = B + 2 * CH
BASE_LEN = NCLS_PAD + 64


def _ema_kernel(feat_hbm, lbl_hbm, protos_hbm, out_hbm,
                protos_v, lblv, cntv, matv, basev, curv, posb, valb,
                idx0, idx1, rows0, rows1, cntmat_sh, queue_sh,
                sem0, sem1, qsem0, qsem1):
    sid = lax.axis_index("s")
    wid = sid * 2 + lax.axis_index("c")
    c0 = wid * K
    lane = lax.iota(jnp.int32, 16)
    zeros16 = jnp.zeros((16,), jnp.int32)
    ones16 = jnp.ones((16,), jnp.int32)

    pltpu.sync_copy(lbl_hbm.at[pl.ds(sid * SLICE, SLICE)], lblv)

    def zero_body(i, _):
        cntv[pl.ds(i * 16, 16)] = zeros16
        return _
    lax.fori_loop(0, NCLS_PAD // 16, zero_body, 0)

    def hist_body(i, _):
        lv = lblv[pl.ds(i * 16, 16)]
        plsc.addupdate_scatter(cntv, [lv], ones16)
        return _
    lax.fori_loop(0, NVREG, hist_body, 0)

    pltpu.sync_copy(cntv, cntmat_sh.at[sid])
    plsc.subcore_barrier()
    pltpu.sync_copy(cntmat_sh, matv)

    def colsum_body(j, _):
        tot = zeros16
        pre = zeros16
        for t in range(NT):
            row = matv[t, pl.ds(j * 16, 16)]
            tot = tot + row
            pre = pre + row * (jnp.int32(t) < sid).astype(jnp.int32)
        cntv[pl.ds(j * 16, 16)] = tot
        curv[pl.ds(j * 16, 16)] = pre
        return _
    lax.fori_loop(0, NCLS_PAD // 16, colsum_body, 0)

    def cumsum_body(j, run):
        chv = cntv[pl.ds(j * 16, 16)]
        inc = plsc.cumsum(chv)
        ex = inc - chv + jnp.full((16,), run, jnp.int32)
        basev[pl.ds(j * 16, 16)] = ex
        curv[pl.ds(j * 16, 16)] = curv[pl.ds(j * 16, 16)] + ex
        return run + inc[15]
    total = lax.fori_loop(0, NCLS_PAD // 16, cumsum_body, jnp.int32(0))
    for j in range(NCLS_PAD // 16, BASE_LEN // 16):
        basev[pl.ds(j * 16, 16)] = jnp.full((16,), total, jnp.int32)

    def place_body(i, _):
        lv = lblv[pl.ds(i * 16, 16)]
        cnt, _last = plsc.scan_count(lv)
        g = plsc.load_gather(curv, [lv])
        pos = g + cnt - 1
        r = lax.div(i, 8)
        col = 16 * lax.rem(i, 8)
        posb[r, pl.ds(col, 16)] = pos
        valb[r, pl.ds(col, 16)] = sid * SLICE + i * 16 + lane
        plsc.addupdate_scatter(curv, [lv], ones16)
        return _
    lax.fori_loop(0, NVREG, place_body, 0)

    for jj in range(8):
        pltpu.async_copy(valb.at[jj], queue_sh.at[posb.at[jj]], sem0)
    for jj in range(8):
        pltpu.make_async_copy(valb.at[jj], queue_sh.at[posb.at[jj]],
                              sem0).wait()

    @pl.when(sid == 0)
    def _():
        def zq_body(i, _):
            idx0[pl.ds(i * 16, 16)] = zeros16
            return _
        lax.fori_loop(0, CH // 16, zq_body, 0)
        pltpu.sync_copy(idx0, queue_sh.at[pl.ds(B, CH)])
        pltpu.sync_copy(idx0, queue_sh.at[pl.ds(B + CH, CH)])
    plsc.subcore_barrier()

    pltpu.sync_copy(protos_hbm.at[pl.ds(c0, K)], protos_v)
    bfly = [lax.bitwise_xor(lane, jnp.int32(s)) for s in (8, 4, 2, 1)]
    sems = (sem0, sem1)
    qsems = (qsem0, qsem1)
    idxs = (idx0, idx1)
    rows = (rows0, rows1)

    def off_at(i):
        return basev[pl.ds(c0 + i, 16)][0]

    s0 = off_at(0)
    s1 = off_at(K)
    al = pl.multiple_of(lax.bitwise_and(s0, -16), 16)
    nwin = lax.div(s1 - al + (CH - 1), CH)

    def win_start(t):
        return pl.multiple_of(al + t * CH, 16)

    def idx_copy(t, buf):
        pltpu.async_copy(queue_sh.at[pl.ds(win_start(t), CH)], idxs[buf],
                         qsems[buf])

    def idx_wait(t, buf):
        pltpu.make_async_copy(queue_sh.at[pl.ds(win_start(t), CH)],
                              idxs[buf], qsems[buf]).wait()

    @pl.when(nwin > 0)
    def _():
        idx_copy(0, 0)
        idx_wait(0, 0)
        pltpu.async_copy(feat_hbm.at[idxs[0]], rows[0], sems[0])

    @pl.when(nwin > 1)
    def _():
        idx_copy(1, 1)

    def do_window(t, buf, cur_p):
        @pl.when(t < nwin)
        def _():
            pltpu.make_async_copy(feat_hbm.at[idxs[buf]],
                                  rows[buf], sems[buf]).wait()

        @pl.when(t + 1 < nwin)
        def _():
            idx_wait(t + 1, 1 - buf)
            pltpu.async_copy(feat_hbm.at[idxs[1 - buf]], rows[1 - buf],
                             sems[1 - buf])

        @pl.when(t + 2 < nwin)
        def _():
            idx_copy(t + 2, buf)

        base = al + t * CH
        lo = jnp.maximum(s0 - base, 0)
        hi = jnp.minimum(s1 - base, CH)

        def samp_body(r, cur_p):
            j = base + r

            def sw_true(ci, bnd, *p):
                for v in range(NV):
                    protos_v[ci, pl.ds(v * 16, 16)] = p[v]
                ci = lax.while_loop(lambda a: off_at(a + 1) <= j,
                                    lambda a: a + 1, ci)
                return (ci, off_at(ci + 1)) + tuple(
                    protos_v[ci, pl.ds(v * 16, 16)] for v in range(NV))

            def sw_false(ci, bnd, *p):
                return (ci, bnd) + tuple(p)

            cur_p = lax.cond(j >= cur_p[1], sw_true, sw_false, *cur_p)
            p = cur_p[2:]
            q = [PROTO_M * p[v]
                 + (1.0 - PROTO_M) * rows[buf][r, pl.ds(v * 16, 16)]
                 for v in range(NV)]
            d = [q[2 * v] * q[2 * v] + q[2 * v + 1] * q[2 * v + 1]
                 for v in range(NV // 2)]
            d = [d[0] + d[1], d[2] + d[3]]
            acc = d[0] + d[1]
            n2 = plsc.cumsum(acc)[15]
            n2v = jnp.maximum(jnp.full((16,), n2, jnp.float32),
                              jnp.float32(1e-24))
            y = plsc.bitcast(
                jnp.int32(0x5F3759DF)
                - lax.shift_right_arithmetic(
                    plsc.bitcast(n2v, jnp.int32), jnp.int32(1)),
                jnp.float32)
            h = jnp.float32(0.5) * n2v
            for _ in range(2):
                y = y * (jnp.float32(1.5) - h * y * y)
            return cur_p[:2] + tuple(q[v] * y for v in range(NV))

        return lax.fori_loop(lo, hi, samp_body, cur_p)

    def pair_body(g, cur_p):
        cur_p = do_window(2 * g, 0, cur_p)
        cur_p = do_window(2 * g + 1, 1, cur_p)
        return cur_p

    ci0 = lax.while_loop(
        lambda a: jnp.logical_and(a < K, off_at(a + 1) <= s0),
        lambda a: a + 1, 0)
    ci0 = jnp.minimum(ci0, K - 1)
    cur_p = (ci0, off_at(ci0 + 1)) + tuple(
        protos_v[ci0, pl.ds(v * 16, 16)] for v in range(NV))
    cur_p = lax.fori_loop(0, lax.div(nwin + 1, 2), pair_body, cur_p)

    @pl.when(s1 > s0)
    def _():
        for v in range(NV):
            protos_v[cur_p[0], pl.ds(v * 16, 16)] = cur_p[2 + v]

    pltpu.sync_copy(protos_v, out_hbm.at[pl.ds(c0, K)])


def _loss_kernel(p_ref, o_ref):
    p = p_ref[...]
    s = lax.dot_general(p, p, (((1,), (1,)), ((), ())),
                        preferred_element_type=jnp.float32) * (1.0 / TEMP)
    row = lax.broadcasted_iota(jnp.int32, (NCLS_PAD, NCLS_PAD), 0)
    col = lax.broadcasted_iota(jnp.int32, (NCLS_PAD, NCLS_PAD), 1)
    neg = jnp.logical_and(col < N_CLS, col != row)
    e = jnp.where(neg, jnp.exp(s), 0.0)
    rs = jnp.sum(e, axis=1)
    mpn = jnp.log(rs / jnp.float32(N_CLS - 1))
    rvalid = lax.broadcasted_iota(jnp.int32, (NCLS_PAD, 1), 0) < N_CLS
    total = jnp.sum(jnp.where(rvalid[:, 0], mpn, 0.0))
    loss = (TEMP / BASE_TEMP) * total / jnp.float32(N_CLS)
    o_ref[...] = jnp.full((8, 128), loss, jnp.float32)


@jax.jit
def kernel(features, labels, prototypes):
    labels = labels.astype(jnp.int32)
    protos_pad = jnp.pad(prototypes, ((0, NCLS_PAD - N_CLS), (0, 0)))

    ema = pl.kernel(
        _ema_kernel,
        out_type=jax.ShapeDtypeStruct((NCLS_PAD, D), jnp.float32),
        mesh=plsc.VectorSubcoreMesh(core_axis_name="c", subcore_axis_name="s"),
        compiler_params=pltpu.CompilerParams(needs_layout_passes=False),
        scratch_types=[
            pltpu.VMEM((K, D), jnp.float32),
            pltpu.VMEM((SLICE,), jnp.int32),
            pltpu.VMEM((NCLS_PAD,), jnp.int32),
            pltpu.VMEM((NT, NCLS_PAD), jnp.int32),
            pltpu.VMEM((BASE_LEN,), jnp.int32),
            pltpu.VMEM((NCLS_PAD,), jnp.int32),
            pltpu.VMEM((8, 128), jnp.int32),
            pltpu.VMEM((8, 128), jnp.int32),
            pltpu.VMEM((CH,), jnp.int32),
            pltpu.VMEM((CH,), jnp.int32),
            pltpu.VMEM((CH, D), jnp.float32),
            pltpu.VMEM((CH, D), jnp.float32),
            pltpu.VMEM_SHARED((NT, NCLS_PAD), jnp.int32),
            pltpu.VMEM_SHARED((QLEN,), jnp.int32),
            pltpu.SemaphoreType.DMA,
            pltpu.SemaphoreType.DMA,
            pltpu.SemaphoreType.DMA,
            pltpu.SemaphoreType.DMA,
        ],
    )
    protos_new = ema(features, labels, protos_pad)

    loss2d = pl.pallas_call(
        _loss_kernel,
        out_shape=jax.ShapeDtypeStruct((8, 128), jnp.float32),
    )(protos_new)
    return loss2d[0, 0]
